# Initial kernel scaffold; baseline (speedup 1.0000x reference)
import jax
import jax.numpy as jnp
from jax.experimental import pallas as pl


def kernel(matrix):
    raise NotImplementedError("write your pallas kernel here")


# speedup vs baseline: 43.7006x; 43.7006x over previous
import functools

import jax
import jax.numpy as jnp
from jax import lax
from jax.experimental import pallas as pl
from jax.experimental.pallas import tpu as pltpu
from jax.experimental.pallas import tpu_sc as plsc

MARGIN = 0.2
N = 4096
NC, NS, L = 2, 16, 16
NW = NC * NS
ROWS_PER_W = N // NW
R = 8
NCHUNK = ROWS_PER_W // R
NEG_INF = float("-inf")

_mesh = plsc.VectorSubcoreMesh(core_axis_name="c", subcore_axis_name="s")


@functools.partial(
    pl.kernel,
    out_type=jax.ShapeDtypeStruct((NW, L), jnp.float32),
    mesh=_mesh,
    compiler_params=pltpu.CompilerParams(use_tc_tiling_on_sc=False,
                                         needs_layout_passes=False),
    scratch_types=[
        pltpu.VMEM((R, N), jnp.float32),
        pltpu.VMEM((R, N), jnp.float32),
        pltpu.VMEM((L,), jnp.float32),
        pltpu.SemaphoreType.DMA,
        pltpu.SemaphoreType.DMA,
    ],
)
def _loss_partials(mat_hbm, out_hbm, buf0, buf1, obuf, sem0, sem1):
    wid = lax.axis_index("s") * NC + lax.axis_index("c")
    base = wid * ROWS_PER_W
    bufs = (buf0, buf1)
    sems = (sem0, sem1)

    pltpu.async_copy(mat_hbm.at[pl.ds(base, R)], buf0, sem0)
    pltpu.async_copy(mat_hbm.at[pl.ds(base + R, R)], buf1, sem1)

    def row_loss(buf, r, col):
        r_vec = jnp.full((L,), r, jnp.int32)
        c_vec = jnp.full((L,), col, jnp.int32)
        pos = jnp.max(plsc.load_gather(buf, [r_vec, c_vec]))
        lane0 = lax.iota(jnp.int32, L) == 0
        plsc.store_scatter(buf, [r_vec, c_vec],
                           jnp.full((L,), NEG_INF, jnp.float32), mask=lane0)

        def inner(k, accs):
            a0, a1, a2, a3 = accs
            c0 = k * 64
            a0 = jnp.maximum(a0, buf[r, pl.ds(c0, L)])
            a1 = jnp.maximum(a1, buf[r, pl.ds(c0 + 16, L)])
            a2 = jnp.maximum(a2, buf[r, pl.ds(c0 + 32, L)])
            a3 = jnp.maximum(a3, buf[r, pl.ds(c0 + 48, L)])
            return (a0, a1, a2, a3)

        neg_fill = jnp.full((L,), NEG_INF, jnp.float32)
        accs = lax.fori_loop(0, N // 64, inner,
                             (neg_fill, neg_fill, neg_fill, neg_fill))
        m = jnp.max(jnp.maximum(jnp.maximum(accs[0], accs[1]),
                                jnp.maximum(accs[2], accs[3])))
        return jnp.maximum(m - pos + MARGIN, 0.0)

    def outer(t, loss):
        for b in range(2):
            c = t * 2 + b
            row0 = base + c * R
            pltpu.make_async_copy(
                mat_hbm.at[pl.ds(row0, R)], bufs[b], sems[b]).wait()
            for r in range(R):
                loss = loss + row_loss(bufs[b], r, row0 + r)

            @pl.when(c + 2 < NCHUNK)
            def _():
                pltpu.async_copy(
                    mat_hbm.at[pl.ds(row0 + 2 * R, R)], bufs[b], sems[b])
        return loss

    loss = lax.fori_loop(0, NCHUNK // 2, outer, jnp.float32(0.0))
    obuf[...] = jnp.broadcast_to(loss, (L,))
    pltpu.sync_copy(obuf, out_hbm.at[wid])


def kernel(matrix):
    partials = _loss_partials(matrix)
    return jnp.sum(partials[:, 0])


# speedup vs baseline: 48.1139x; 1.1010x over previous
import functools

import jax
import jax.numpy as jnp
from jax import lax
from jax.experimental import pallas as pl
from jax.experimental.pallas import tpu as pltpu
from jax.experimental.pallas import tpu_sc as plsc

MARGIN = 0.2
N = 4096
NC, NS, L = 2, 16, 16
NW = NC * NS
ROWS_PER_W = N // NW
R = 4
NBUF = 4
NCHUNK = ROWS_PER_W // R
SLICES = 8
COLS_PER_IT = SLICES * L
NEG_INF = float("-inf")

_mesh = plsc.VectorSubcoreMesh(core_axis_name="c", subcore_axis_name="s")


@functools.partial(
    pl.kernel,
    out_type=jax.ShapeDtypeStruct((NW, L), jnp.float32),
    mesh=_mesh,
    compiler_params=pltpu.CompilerParams(use_tc_tiling_on_sc=False,
                                         needs_layout_passes=False),
    scratch_types=[
        [pltpu.VMEM((R, N), jnp.float32) for _ in range(NBUF)],
        pltpu.VMEM((L,), jnp.float32),
        [pltpu.SemaphoreType.DMA for _ in range(NBUF)],
    ],
)
def _loss_partials(mat_hbm, out_hbm, bufs, obuf, sems):
    wid = lax.axis_index("s") * NC + lax.axis_index("c")
    base = wid * ROWS_PER_W

    for b in range(NBUF):
        pltpu.async_copy(mat_hbm.at[pl.ds(base + b * R, R)], bufs[b], sems[b])

    lane0 = lax.iota(jnp.int32, L) == 0
    neg_fill = jnp.full((L,), NEG_INF, jnp.float32)

    def chunk_loss(buf, row0, loss):
        poss = []
        for r in range(R):
            r_vec = jnp.full((L,), r, jnp.int32)
            c_vec = jnp.full((L,), row0 + r, jnp.int32)
            poss.append(jnp.max(plsc.load_gather(buf, [r_vec, c_vec])))
            plsc.store_scatter(buf, [r_vec, c_vec],
                               jnp.full((L,), NEG_INF, jnp.float32),
                               mask=lane0)

        def inner(k, accs):
            c0 = k * COLS_PER_IT
            out = []
            for r in range(R):
                vs = [buf[r, pl.ds(c0 + s * L, L)] for s in range(SLICES)]
                while len(vs) > 1:
                    vs = [jnp.maximum(vs[i], vs[i + 1])
                          for i in range(0, len(vs), 2)]
                out.append(jnp.maximum(accs[r], vs[0]))
            return tuple(out)

        accs = lax.fori_loop(0, N // COLS_PER_IT, inner, (neg_fill,) * R)
        for r in range(R):
            loss = loss + jnp.maximum(jnp.max(accs[r]) - poss[r] + MARGIN,
                                      0.0)
        return loss

    def outer(t, loss):
        for b in range(NBUF):
            c = t * NBUF + b
            row0 = base + c * R
            pltpu.make_async_copy(
                mat_hbm.at[pl.ds(row0, R)], bufs[b], sems[b]).wait()
            loss = chunk_loss(bufs[b], row0, loss)

            @pl.when(c + NBUF < NCHUNK)
            def _():
                pltpu.async_copy(
                    mat_hbm.at[pl.ds(row0 + NBUF * R, R)], bufs[b], sems[b])
        return loss

    loss = lax.fori_loop(0, NCHUNK // NBUF, outer, jnp.float32(0.0))
    obuf[...] = jnp.broadcast_to(loss, (L,))
    pltpu.sync_copy(obuf, out_hbm.at[wid])


def kernel(matrix):
    partials = _loss_partials(matrix)
    return jnp.sum(partials[:, 0])


# speedup vs baseline: 90.5747x; 1.8825x over previous
import functools

import jax
import jax.numpy as jnp
from jax import lax
from jax.experimental import pallas as pl
from jax.experimental.pallas import tpu as pltpu
from jax.experimental.pallas import tpu_sc as plsc

MARGIN = 0.2
N = 4096
NC, NS, L = 2, 16, 16
NW = NC * NS
ROWS_PER_W = N // NW
R = 8
NBUF = 2
NCHUNK = ROWS_PER_W // R
SLICES = 8
COLS_PER_IT = SLICES * L
NEG_INF = float("-inf")

_mesh = plsc.VectorSubcoreMesh(core_axis_name="c", subcore_axis_name="s")


@functools.partial(
    pl.kernel,
    out_type=jax.ShapeDtypeStruct((NW, L), jnp.float32),
    mesh=_mesh,
    compiler_params=pltpu.CompilerParams(needs_layout_passes=False),
    scratch_types=[
        [pltpu.VMEM((R, N), jnp.float32) for _ in range(NBUF)],
        pltpu.VMEM((L,), jnp.float32),
        [pltpu.SemaphoreType.DMA for _ in range(NBUF)],
    ],
)
def _loss_partials(mat_hbm, out_hbm, bufs, obuf, sems):
    wid = lax.axis_index("s") * NC + lax.axis_index("c")
    base = wid * ROWS_PER_W

    for b in range(NBUF):
        pltpu.async_copy(mat_hbm.at[pl.ds(base + b * R, R)], bufs[b], sems[b])

    lanes = lax.iota(jnp.int32, L)
    neg_fill = jnp.full((L,), NEG_INF, jnp.float32)

    def chunk_loss(buf, row0, loss):
        poss = []
        for r in range(R):
            col = row0 + r
            sb = (col // L) * L
            tgt = col - sb
            v = buf[r, pl.ds(sb, L)]
            m = lanes == tgt
            poss.append(jnp.max(jnp.where(m, v, neg_fill)))
            buf[r, pl.ds(sb, L)] = jnp.where(m, neg_fill, v)

        def inner(k, accs):
            c0 = k * COLS_PER_IT
            out = []
            for r in range(R):
                vs = [buf[r, pl.ds(c0 + s * L, L)] for s in range(SLICES)]
                while len(vs) > 1:
                    vs = [jnp.maximum(vs[i], vs[i + 1])
                          for i in range(0, len(vs), 2)]
                out.append(jnp.maximum(accs[r], vs[0]))
            return tuple(out)

        accs = lax.fori_loop(0, N // COLS_PER_IT, inner, (neg_fill,) * R)
        for r in range(R):
            loss = loss + jnp.maximum(jnp.max(accs[r]) - poss[r] + MARGIN,
                                      0.0)
        return loss

    def outer(t, loss):
        for b in range(NBUF):
            c = t * NBUF + b
            row0 = base + c * R
            pltpu.make_async_copy(
                mat_hbm.at[pl.ds(row0, R)], bufs[b], sems[b]).wait()
            loss = chunk_loss(bufs[b], row0, loss)

            @pl.when(c + NBUF < NCHUNK)
            def _():
                pltpu.async_copy(
                    mat_hbm.at[pl.ds(row0 + NBUF * R, R)], bufs[b], sems[b])
        return loss

    loss = lax.fori_loop(0, NCHUNK // NBUF, outer, jnp.float32(0.0))
    obuf[...] = jnp.broadcast_to(loss, (L,))
    pltpu.sync_copy(obuf, out_hbm.at[wid])


def kernel(matrix):
    partials = _loss_partials(matrix)
    return jnp.sum(partials[:, 0])


# speedup vs baseline: 95.9147x; 1.0590x over previous
import functools

import jax
import jax.numpy as jnp
from jax import lax
from jax.experimental import pallas as pl
from jax.experimental.pallas import tpu as pltpu
from jax.experimental.pallas import tpu_sc as plsc

MARGIN = 0.2
N = 4096
NC, NS, L = 2, 16, 16
NW = NC * NS
ROWS_PER_W = N // NW
R = 8
NBUF = 3
NCHUNK = ROWS_PER_W // R
SLICES = 8
COLS_PER_IT = SLICES * L
NEG_INF = float("-inf")

_mesh = plsc.VectorSubcoreMesh(core_axis_name="c", subcore_axis_name="s")


@functools.partial(
    pl.kernel,
    out_type=jax.ShapeDtypeStruct((NW, L), jnp.float32),
    mesh=_mesh,
    compiler_params=pltpu.CompilerParams(needs_layout_passes=False),
    scratch_types=[
        [pltpu.VMEM((R, N), jnp.float32) for _ in range(NBUF)],
        pltpu.VMEM((L,), jnp.float32),
        [pltpu.SemaphoreType.DMA for _ in range(NBUF)],
    ],
)
def _loss_partials(mat_hbm, out_hbm, bufs, obuf, sems):
    wid = lax.axis_index("s") * NC + lax.axis_index("c")
    base = wid * ROWS_PER_W

    for b in range(NBUF):
        pltpu.async_copy(mat_hbm.at[pl.ds(base + b * R, R)], bufs[b], sems[b])

    lanes = lax.iota(jnp.int32, L)
    neg_fill = jnp.full((L,), NEG_INF, jnp.float32)

    def chunk_loss(buf, row0, loss):
        poss = []
        for r in range(R):
            col = row0 + r
            sb = (col // L) * L
            tgt = col - sb
            v = buf[r, pl.ds(sb, L)]
            m = lanes == tgt
            poss.append(jnp.max(jnp.where(m, v, neg_fill)))
            buf[r, pl.ds(sb, L)] = jnp.where(m, neg_fill, v)

        def inner(k, accs):
            c0 = k * COLS_PER_IT
            out = []
            for r in range(R):
                vs = [buf[r, pl.ds(c0 + s * L, L)] for s in range(SLICES)]
                while len(vs) > 1:
                    vs = [jnp.maximum(vs[i], vs[i + 1])
                          for i in range(0, len(vs), 2)]
                out.append(jnp.maximum(accs[r], vs[0]))
            return tuple(out)

        accs = lax.fori_loop(0, N // COLS_PER_IT, inner, (neg_fill,) * R)
        for r in range(R):
            loss = loss + jnp.maximum(jnp.max(accs[r]) - poss[r] + MARGIN,
                                      0.0)
        return loss

    def outer(t, loss):
        for b in range(NBUF):
            c = t * NBUF + b
            row0 = base + c * R
            pltpu.make_async_copy(
                mat_hbm.at[pl.ds(row0, R)], bufs[b], sems[b]).wait()
            loss = chunk_loss(bufs[b], row0, loss)

            @pl.when(c + NBUF < NCHUNK)
            def _():
                pltpu.async_copy(
                    mat_hbm.at[pl.ds(row0 + NBUF * R, R)], bufs[b], sems[b])
        return loss

    loss = lax.fori_loop(0, NCHUNK // NBUF, outer, jnp.float32(0.0))
    for c in range((NCHUNK // NBUF) * NBUF, NCHUNK):
        b = c % NBUF
        row0 = base + c * R
        pltpu.make_async_copy(
            mat_hbm.at[pl.ds(row0, R)], bufs[b], sems[b]).wait()
        loss = chunk_loss(bufs[b], row0, loss)
    obuf[...] = jnp.broadcast_to(loss, (L,))
    pltpu.sync_copy(obuf, out_hbm.at[wid])


def kernel(matrix):
    partials = _loss_partials(matrix)
    return jnp.sum(partials[:, 0])


# speedup vs baseline: 100.6801x; 1.0497x over previous
import functools

import jax
import jax.numpy as jnp
from jax import lax
from jax.experimental import pallas as pl
from jax.experimental.pallas import tpu as pltpu
from jax.experimental.pallas import tpu_sc as plsc

MARGIN = 0.2
N = 4096
TC_ROWS = 1024
TC_BLK = 256
NC, NS, L = 2, 16, 16
NW = NC * NS
SC_ROWS = N - TC_ROWS
ROWS_PER_W = SC_ROWS // NW
R = 8
NBUF = 3
NCHUNK = ROWS_PER_W // R
SLICES = 8
COLS_PER_IT = SLICES * L
NEG_INF = float("-inf")

_mesh = plsc.VectorSubcoreMesh(core_axis_name="c", subcore_axis_name="s")


@functools.partial(
    pl.kernel,
    out_type=jax.ShapeDtypeStruct((NW, L), jnp.float32),
    mesh=_mesh,
    compiler_params=pltpu.CompilerParams(needs_layout_passes=False),
    scratch_types=[
        [pltpu.VMEM((R, N), jnp.float32) for _ in range(NBUF)],
        pltpu.VMEM((L,), jnp.float32),
        [pltpu.SemaphoreType.DMA for _ in range(NBUF)],
    ],
)
def _sc_partials(mat_hbm, out_hbm, bufs, obuf, sems):
    wid = lax.axis_index("s") * NC + lax.axis_index("c")
    base = TC_ROWS + wid * ROWS_PER_W

    for b in range(NBUF):
        pltpu.async_copy(mat_hbm.at[pl.ds(base + b * R, R)], bufs[b], sems[b])

    lanes = lax.iota(jnp.int32, L)
    neg_fill = jnp.full((L,), NEG_INF, jnp.float32)

    def chunk_loss(buf, row0, loss):
        poss = []
        for r in range(R):
            col = row0 + r
            sb = (col // L) * L
            tgt = col - sb
            v = buf[r, pl.ds(sb, L)]
            m = lanes == tgt
            poss.append(jnp.max(jnp.where(m, v, neg_fill)))
            buf[r, pl.ds(sb, L)] = jnp.where(m, neg_fill, v)

        def inner(k, accs):
            c0 = k * COLS_PER_IT
            out = []
            for r in range(R):
                vs = [buf[r, pl.ds(c0 + s * L, L)] for s in range(SLICES)]
                while len(vs) > 1:
                    vs = [jnp.maximum(vs[i], vs[i + 1])
                          for i in range(0, len(vs), 2)]
                out.append(jnp.maximum(accs[r], vs[0]))
            return tuple(out)

        accs = lax.fori_loop(0, N // COLS_PER_IT, inner, (neg_fill,) * R)
        for r in range(R):
            loss = loss + jnp.maximum(jnp.max(accs[r]) - poss[r] + MARGIN,
                                      0.0)
        return loss

    def outer(t, loss):
        for b in range(NBUF):
            c = t * NBUF + b
            row0 = base + c * R
            pltpu.make_async_copy(
                mat_hbm.at[pl.ds(row0, R)], bufs[b], sems[b]).wait()
            loss = chunk_loss(bufs[b], row0, loss)

            @pl.when(c + NBUF < NCHUNK)
            def _():
                pltpu.async_copy(
                    mat_hbm.at[pl.ds(row0 + NBUF * R, R)], bufs[b], sems[b])
        return loss

    loss = lax.fori_loop(0, NCHUNK // NBUF, outer, jnp.float32(0.0))
    for c in range((NCHUNK // NBUF) * NBUF, NCHUNK):
        b = c % NBUF
        row0 = base + c * R
        pltpu.make_async_copy(
            mat_hbm.at[pl.ds(row0, R)], bufs[b], sems[b]).wait()
        loss = chunk_loss(bufs[b], row0, loss)
    obuf[...] = jnp.broadcast_to(loss, (L,))
    pltpu.sync_copy(obuf, out_hbm.at[wid])


def _tc_body(mat_ref, out_ref):
    i = pl.program_id(0)
    rows = lax.broadcasted_iota(jnp.int32, (TC_BLK, N), 0) + i * TC_BLK
    cols = lax.broadcasted_iota(jnp.int32, (TC_BLK, N), 1)
    diag = rows == cols
    blk = mat_ref[...]
    pos = jnp.max(jnp.where(diag, blk, NEG_INF), axis=1)
    neg = jnp.max(jnp.where(diag, NEG_INF, blk), axis=1)
    part = jnp.sum(jnp.maximum(neg - pos + MARGIN, 0.0))

    @pl.when(i == 0)
    def _():
        out_ref[0, 0] = 0.0

    out_ref[0, 0] += part


_tc_loss = pl.pallas_call(
    _tc_body,
    grid=(TC_ROWS // TC_BLK,),
    in_specs=[pl.BlockSpec((TC_BLK, N), lambda i: (i, 0))],
    out_specs=pl.BlockSpec(memory_space=pltpu.SMEM),
    out_shape=jax.ShapeDtypeStruct((1, 1), jnp.float32),
)


def kernel(matrix):
    sc = _sc_partials(matrix)
    tc = _tc_loss(matrix)
    return tc[0, 0] + jnp.sum(sc[:, 0])


# speedup vs baseline: 101.9262x; 1.0124x over previous
import functools

import jax
import jax.numpy as jnp
from jax import lax
from jax.experimental import pallas as pl
from jax.experimental.pallas import tpu as pltpu
from jax.experimental.pallas import tpu_sc as plsc

MARGIN = 0.2
N = 4096
TC_ROWS = 1536
TC_BLK = 256
NC, NS, L = 2, 16, 16
NW = NC * NS
SC_ROWS = N - TC_ROWS
ROWS_PER_W = SC_ROWS // NW
R = 8
NBUF = 3
NCHUNK = ROWS_PER_W // R
SLICES = 4
COLS_PER_IT = SLICES * L
NEG_INF = float("-inf")

_mesh = plsc.VectorSubcoreMesh(core_axis_name="c", subcore_axis_name="s")


@functools.partial(
    pl.kernel,
    out_type=jax.ShapeDtypeStruct((NW, L), jnp.float32),
    mesh=_mesh,
    compiler_params=pltpu.CompilerParams(needs_layout_passes=False),
    scratch_types=[
        [pltpu.VMEM((R, N), jnp.float32) for _ in range(NBUF)],
        pltpu.VMEM((L,), jnp.float32),
        [pltpu.SemaphoreType.DMA for _ in range(NBUF)],
    ],
)
def _sc_partials(mat_hbm, out_hbm, bufs, obuf, sems):
    wid = lax.axis_index("s") * NC + lax.axis_index("c")
    base = TC_ROWS + wid * ROWS_PER_W

    for b in range(NBUF):
        pltpu.async_copy(mat_hbm.at[pl.ds(base + b * R, R)], bufs[b], sems[b])

    lanes = lax.iota(jnp.int32, L)
    neg_fill = jnp.full((L,), NEG_INF, jnp.float32)

    def chunk_loss(buf, row0, loss):
        poss = []
        for r in range(R):
            col = row0 + r
            sb = (col // L) * L
            tgt = col - sb
            v = buf[r, pl.ds(sb, L)]
            m = lanes == tgt
            poss.append(jnp.max(jnp.where(m, v, neg_fill)))
            buf[r, pl.ds(sb, L)] = jnp.where(m, neg_fill, v)

        def inner(k, accs):
            c0 = k * COLS_PER_IT
            out = []
            for r in range(R):
                vs = [buf[r, pl.ds(c0 + s * L, L)] for s in range(SLICES)]
                while len(vs) > 1:
                    vs = [jnp.maximum(vs[i], vs[i + 1])
                          for i in range(0, len(vs), 2)]
                out.append(jnp.maximum(accs[r], vs[0]))
            return tuple(out)

        accs = lax.fori_loop(0, N // COLS_PER_IT, inner, (neg_fill,) * R)
        for r in range(R):
            loss = loss + jnp.maximum(jnp.max(accs[r]) - poss[r] + MARGIN,
                                      0.0)
        return loss

    def outer(t, loss):
        for b in range(NBUF):
            c = t * NBUF + b
            row0 = base + c * R
            pltpu.make_async_copy(
                mat_hbm.at[pl.ds(row0, R)], bufs[b], sems[b]).wait()
            loss = chunk_loss(bufs[b], row0, loss)

            @pl.when(c + NBUF < NCHUNK)
            def _():
                pltpu.async_copy(
                    mat_hbm.at[pl.ds(row0 + NBUF * R, R)], bufs[b], sems[b])
        return loss

    loss = lax.fori_loop(0, NCHUNK // NBUF, outer, jnp.float32(0.0))
    for c in range((NCHUNK // NBUF) * NBUF, NCHUNK):
        b = c % NBUF
        row0 = base + c * R
        pltpu.make_async_copy(
            mat_hbm.at[pl.ds(row0, R)], bufs[b], sems[b]).wait()
        loss = chunk_loss(bufs[b], row0, loss)
    obuf[...] = jnp.broadcast_to(loss, (L,))
    pltpu.sync_copy(obuf, out_hbm.at[wid])


def _tc_body(mat_ref, out_ref):
    i = pl.program_id(0)
    rl = lax.broadcasted_iota(jnp.int32, (TC_BLK, TC_BLK), 0)
    cl = lax.broadcasted_iota(jnp.int32, (TC_BLK, TC_BLK), 1)
    dmask = rl == cl
    sub = mat_ref[:, pl.ds(i * TC_BLK, TC_BLK)]
    pos = jnp.max(jnp.where(dmask, sub, NEG_INF), axis=1)
    mat_ref[:, pl.ds(i * TC_BLK, TC_BLK)] = jnp.where(dmask, NEG_INF, sub)
    neg = jnp.max(mat_ref[...], axis=1)
    part = jnp.sum(jnp.maximum(neg - pos + MARGIN, 0.0))

    @pl.when(i == 0)
    def _():
        out_ref[0, 0] = 0.0

    out_ref[0, 0] += part


_tc_loss = pl.pallas_call(
    _tc_body,
    grid=(TC_ROWS // TC_BLK,),
    in_specs=[pl.BlockSpec((TC_BLK, N), lambda i: (i, 0))],
    out_specs=pl.BlockSpec(memory_space=pltpu.SMEM),
    out_shape=jax.ShapeDtypeStruct((1, 1), jnp.float32),
)


def kernel(matrix):
    sc = _sc_partials(matrix)
    tc = _tc_loss(matrix)
    return tc[0, 0] + jnp.sum(sc[:, 0])


# speedup vs baseline: 102.3097x; 1.0038x over previous
import functools

import jax
import jax.numpy as jnp
from jax import lax
from jax.experimental import pallas as pl
from jax.experimental.pallas import tpu as pltpu
from jax.experimental.pallas import tpu_sc as plsc

MARGIN = 0.2
N = 4096
TC_ROWS = 2048
TC_BLK = 256
NC, NS, L = 2, 16, 16
NW = NC * NS
SC_ROWS = N - TC_ROWS
ROWS_PER_W = SC_ROWS // NW
R = 8
NBUF = 3
NCHUNK = ROWS_PER_W // R
SLICES = 8
COLS_PER_IT = SLICES * L
NEG_INF = float("-inf")

_mesh = plsc.VectorSubcoreMesh(core_axis_name="c", subcore_axis_name="s")


@functools.partial(
    pl.kernel,
    out_type=jax.ShapeDtypeStruct((NW, L), jnp.float32),
    mesh=_mesh,
    compiler_params=pltpu.CompilerParams(needs_layout_passes=False),
    scratch_types=[
        [pltpu.VMEM((R, N), jnp.float32) for _ in range(NBUF)],
        pltpu.VMEM((L,), jnp.float32),
        [pltpu.SemaphoreType.DMA for _ in range(NBUF)],
    ],
)
def _sc_partials(mat_hbm, out_hbm, bufs, obuf, sems):
    wid = lax.axis_index("s") * NC + lax.axis_index("c")
    base = TC_ROWS + wid * ROWS_PER_W

    for b in range(NBUF):
        pltpu.async_copy(mat_hbm.at[pl.ds(base + b * R, R)], bufs[b], sems[b])

    lanes = lax.iota(jnp.int32, L)
    neg_fill = jnp.full((L,), NEG_INF, jnp.float32)

    def chunk_loss(buf, row0, loss):
        poss = []
        for r in range(R):
            col = row0 + r
            sb = (col // L) * L
            tgt = col - sb
            v = buf[r, pl.ds(sb, L)]
            m = lanes == tgt
            poss.append(jnp.max(jnp.where(m, v, neg_fill)))
            buf[r, pl.ds(sb, L)] = jnp.where(m, neg_fill, v)

        def inner(k, accs):
            c0 = k * COLS_PER_IT
            out = []
            for r in range(R):
                vs = [buf[r, pl.ds(c0 + s * L, L)] for s in range(SLICES)]
                while len(vs) > 1:
                    vs = [jnp.maximum(vs[i], vs[i + 1])
                          for i in range(0, len(vs), 2)]
                out.append(jnp.maximum(accs[r], vs[0]))
            return tuple(out)

        accs = lax.fori_loop(0, N // COLS_PER_IT, inner, (neg_fill,) * R)
        for r in range(R):
            loss = loss + jnp.maximum(jnp.max(accs[r]) - poss[r] + MARGIN,
                                      0.0)
        return loss

    def outer(t, loss):
        for b in range(NBUF):
            c = t * NBUF + b
            row0 = base + c * R
            pltpu.make_async_copy(
                mat_hbm.at[pl.ds(row0, R)], bufs[b], sems[b]).wait()
            loss = chunk_loss(bufs[b], row0, loss)

            @pl.when(c + NBUF < NCHUNK)
            def _():
                pltpu.async_copy(
                    mat_hbm.at[pl.ds(row0 + NBUF * R, R)], bufs[b], sems[b])
        return loss

    loss = lax.fori_loop(0, NCHUNK // NBUF, outer, jnp.float32(0.0))
    for c in range((NCHUNK // NBUF) * NBUF, NCHUNK):
        b = c % NBUF
        row0 = base + c * R
        pltpu.make_async_copy(
            mat_hbm.at[pl.ds(row0, R)], bufs[b], sems[b]).wait()
        loss = chunk_loss(bufs[b], row0, loss)
    obuf[...] = jnp.broadcast_to(loss, (L,))
    pltpu.sync_copy(obuf, out_hbm.at[wid])


def _tc_body(mat_ref, out_ref):
    i = pl.program_id(0)
    rl = lax.broadcasted_iota(jnp.int32, (TC_BLK, TC_BLK), 0)
    cl = lax.broadcasted_iota(jnp.int32, (TC_BLK, TC_BLK), 1)
    dmask = rl == cl
    sub = mat_ref[:, pl.ds(i * TC_BLK, TC_BLK)]
    pos = jnp.max(jnp.where(dmask, sub, NEG_INF), axis=1)
    mat_ref[:, pl.ds(i * TC_BLK, TC_BLK)] = jnp.where(dmask, NEG_INF, sub)
    neg = jnp.max(mat_ref[...], axis=1)
    part = jnp.sum(jnp.maximum(neg - pos + MARGIN, 0.0))

    @pl.when(i == 0)
    def _():
        out_ref[0, 0] = 0.0

    out_ref[0, 0] += part


_tc_loss = pl.pallas_call(
    _tc_body,
    grid=(TC_ROWS // TC_BLK,),
    in_specs=[pl.BlockSpec((TC_BLK, N), lambda i: (i, 0))],
    out_specs=pl.BlockSpec(memory_space=pltpu.SMEM),
    out_shape=jax.ShapeDtypeStruct((1, 1), jnp.float32),
)


def kernel(matrix):
    sc = _sc_partials(matrix)
    tc = _tc_loss(matrix)
    return tc[0, 0] + jnp.sum(sc[:, 0])


# speedup vs baseline: 107.3551x; 1.0493x over previous
import functools

import jax
import jax.numpy as jnp
from jax import lax
from jax.experimental import pallas as pl
from jax.experimental.pallas import tpu as pltpu
from jax.experimental.pallas import tpu_sc as plsc

MARGIN = 0.2
N = 4096
TC_ROWS = 2048
TC_BLK = 256
NC, NS, L = 2, 16, 16
NW = NC * NS
SC_ROWS = N - TC_ROWS
ROWS_PER_W = SC_ROWS // NW
R = 8
NBUF = 3
NCHUNK = ROWS_PER_W // R
SLICES = 8
COLS_PER_IT = SLICES * L
NEG_INF = float("-inf")

_mesh = plsc.VectorSubcoreMesh(core_axis_name="c", subcore_axis_name="s")


@functools.partial(
    pl.kernel,
    out_type=jax.ShapeDtypeStruct((NW, L), jnp.float32),
    mesh=_mesh,
    compiler_params=pltpu.CompilerParams(needs_layout_passes=False),
    scratch_types=[
        pltpu.VMEM((NBUF * R, N), jnp.float32),
        pltpu.VMEM((L,), jnp.float32),
        [pltpu.SemaphoreType.DMA for _ in range(NBUF)],
    ],
)
def _sc_partials(mat_hbm, out_hbm, buf, obuf, sems):
    wid = lax.axis_index("s") * NC + lax.axis_index("c")
    base = TC_ROWS + wid * ROWS_PER_W

    for b in range(NBUF):
        pltpu.async_copy(mat_hbm.at[pl.ds(base + b * R, R)],
                         buf.at[pl.ds(b * R, R)], sems[b])

    lanes = lax.iota(jnp.int32, L)
    neg_fill = jnp.full((L,), NEG_INF, jnp.float32)

    def body(c, loss):
        bsel = lax.rem(c, NBUF)
        boff = bsel * R
        row0 = base + c * R
        for b in range(NBUF):
            @pl.when(bsel == b)
            def _():
                pltpu.make_async_copy(
                    mat_hbm.at[pl.ds(row0, R)],
                    buf.at[pl.ds(b * R, R)], sems[b]).wait()

        poss = []
        for r in range(R):
            col = row0 + r
            sb = (col // L) * L
            tgt = col - sb
            v = buf[boff + r, pl.ds(sb, L)]
            m = lanes == tgt
            poss.append(jnp.max(jnp.where(m, v, neg_fill)))
            buf[boff + r, pl.ds(sb, L)] = jnp.where(m, neg_fill, v)

        def inner(k, accs):
            c0 = k * COLS_PER_IT
            out = []
            for r in range(R):
                vs = [buf[boff + r, pl.ds(c0 + s * L, L)]
                      for s in range(SLICES)]
                while len(vs) > 1:
                    vs = [jnp.maximum(vs[i], vs[i + 1])
                          for i in range(0, len(vs), 2)]
                out.append(jnp.maximum(accs[r], vs[0]))
            return tuple(out)

        accs = lax.fori_loop(0, N // COLS_PER_IT, inner, (neg_fill,) * R)
        for r in range(R):
            loss = loss + jnp.maximum(jnp.max(accs[r]) - poss[r] + MARGIN,
                                      0.0)

        for b in range(NBUF):
            @pl.when((bsel == b) & (c + NBUF < NCHUNK))
            def _():
                pltpu.async_copy(
                    mat_hbm.at[pl.ds(row0 + NBUF * R, R)],
                    buf.at[pl.ds(b * R, R)], sems[b])
        return loss

    loss = lax.fori_loop(0, NCHUNK, body, jnp.float32(0.0))
    obuf[...] = jnp.broadcast_to(loss, (L,))
    pltpu.sync_copy(obuf, out_hbm.at[wid])


def _tc_body(mat_ref, out_ref):
    i = pl.program_id(0)
    rl = lax.broadcasted_iota(jnp.int32, (TC_BLK, TC_BLK), 0)
    cl = lax.broadcasted_iota(jnp.int32, (TC_BLK, TC_BLK), 1)
    dmask = rl == cl
    sub = mat_ref[:, pl.ds(i * TC_BLK, TC_BLK)]
    pos = jnp.max(jnp.where(dmask, sub, NEG_INF), axis=1)
    mat_ref[:, pl.ds(i * TC_BLK, TC_BLK)] = jnp.where(dmask, NEG_INF, sub)
    neg = jnp.max(mat_ref[...], axis=1)
    part = jnp.sum(jnp.maximum(neg - pos + MARGIN, 0.0))

    @pl.when(i == 0)
    def _():
        out_ref[0, 0] = 0.0

    out_ref[0, 0] += part


_tc_loss = pl.pallas_call(
    _tc_body,
    grid=(TC_ROWS // TC_BLK,),
    in_specs=[pl.BlockSpec((TC_BLK, N), lambda i: (i, 0))],
    out_specs=pl.BlockSpec(memory_space=pltpu.SMEM),
    out_shape=jax.ShapeDtypeStruct((1, 1), jnp.float32),
)


def kernel(matrix):
    sc = _sc_partials(matrix)
    tc = _tc_loss(matrix)
    return tc[0, 0] + jnp.sum(sc[:, 0])
